# Initial kernel scaffold; baseline (speedup 1.0000x reference)
import jax
import jax.numpy as jnp
from jax.experimental import pallas as pl


def kernel(x):
    raise NotImplementedError("write your pallas kernel here")


# speedup vs baseline: 4.5605x; 4.5605x over previous
import functools

import jax
import jax.numpy as jnp
from jax.experimental import pallas as pl
from jax.experimental.pallas import tpu as pltpu

EPS = 0.1
ROW_BLOCK = 256


def _cumsum_lanes(v, n):
    lane = jax.lax.broadcasted_iota(jnp.int32, (1, n), 1)
    s = 1
    while s < n:
        shifted = pltpu.roll(v, s, axis=1)
        v = v + jnp.where(lane >= s, shifted, 0)
        s *= 2
    return v


def _zono_kernel(x_ref, out_ref, *, n, row_block):
    eps = EPS
    xv = x_ref[...]
    lo = jnp.maximum(eps - xv, 0.0) * 0.5
    hi = jnp.maximum(xv - (1.0 - eps), 0.0) * 0.5
    center = xv + lo - hi
    err = eps - lo - hi
    cond = err >= 0.0
    cond_i = cond.astype(jnp.int32)
    incl = _cumsum_lanes(cond_i, n)
    rows = jnp.where(cond, incl - cond_i + 1, n + 1)

    i = pl.program_id(0)
    row_ids = i * row_block + jax.lax.broadcasted_iota(
        jnp.int32, (row_block, n), 0
    )
    out = jnp.where(row_ids == rows, err, 0.0)
    out = jnp.where(row_ids == 0, center, out)
    out_ref[...] = out


@jax.jit
def kernel(x):
    C, H, W = x.shape
    n = C * H * W
    n_rows = n + 1
    grid = pl.cdiv(n_rows, ROW_BLOCK)
    xf = x.reshape(1, n)
    z = pl.pallas_call(
        functools.partial(_zono_kernel, n=n, row_block=ROW_BLOCK),
        grid=(grid,),
        in_specs=[pl.BlockSpec((1, n), lambda i: (0, 0))],
        out_specs=pl.BlockSpec((ROW_BLOCK, n), lambda i: (i, 0)),
        out_shape=jax.ShapeDtypeStruct((n_rows, n), x.dtype),
    )(xf)
    return z.reshape(n_rows, C, H, W)


# speedup vs baseline: 4.8807x; 1.0702x over previous
import functools

import jax
import jax.numpy as jnp
from jax.experimental import pallas as pl
from jax.experimental.pallas import tpu as pltpu

EPS = 0.1
ROW_BLOCK = 512


def _cumsum_lanes(v, n):
    lane = jax.lax.broadcasted_iota(jnp.int32, (1, n), 1)
    s = 1
    while s < n:
        shifted = pltpu.roll(v, s, axis=1)
        v = v + jnp.where(lane >= s, shifted, 0)
        s *= 2
    return v


def _zono_kernel(x_ref, out_ref, *, n, row_block):
    eps = EPS
    xv = x_ref[...]
    lo = jnp.maximum(eps - xv, 0.0) * 0.5
    hi = jnp.maximum(xv - (1.0 - eps), 0.0) * 0.5
    center = xv + lo - hi
    err = eps - lo - hi
    cond = err >= 0.0
    cond_i = cond.astype(jnp.int32)
    incl = _cumsum_lanes(cond_i, n)
    rows = jnp.where(cond, incl - cond_i + 1, n + 1)

    i = pl.program_id(0)
    row_ids = i * row_block + jax.lax.broadcasted_iota(
        jnp.int32, (row_block, n), 0
    )
    out = jnp.where(row_ids == rows, err, 0.0)
    out = jnp.where(row_ids == 0, center, out)
    out_ref[...] = out


@jax.jit
def kernel(x):
    C, H, W = x.shape
    n = C * H * W
    n_rows = n + 1
    grid = pl.cdiv(n_rows, ROW_BLOCK)
    xf = x.reshape(1, n)
    z = pl.pallas_call(
        functools.partial(_zono_kernel, n=n, row_block=ROW_BLOCK),
        grid=(grid,),
        in_specs=[pl.BlockSpec((1, n), lambda i: (0, 0))],
        out_specs=pl.BlockSpec((ROW_BLOCK, n), lambda i: (i, 0)),
        out_shape=jax.ShapeDtypeStruct((n_rows, n), x.dtype),
    )(xf)
    return z.reshape(n_rows, C, H, W)


# speedup vs baseline: 4.9328x; 1.0107x over previous
import functools

import jax
import jax.numpy as jnp
from jax.experimental import pallas as pl
from jax.experimental.pallas import tpu as pltpu

EPS = 0.1
ROW_BLOCK = 1024


def _cumsum_lanes(v, n):
    lane = jax.lax.broadcasted_iota(jnp.int32, (1, n), 1)
    s = 1
    while s < n:
        shifted = pltpu.roll(v, s, axis=1)
        v = v + jnp.where(lane >= s, shifted, 0)
        s *= 2
    return v


def _zono_kernel(x_ref, out_ref, *, n, row_block):
    eps = EPS
    xv = x_ref[...]
    lo = jnp.maximum(eps - xv, 0.0) * 0.5
    hi = jnp.maximum(xv - (1.0 - eps), 0.0) * 0.5
    center = xv + lo - hi
    err = eps - lo - hi
    cond = err >= 0.0
    cond_i = cond.astype(jnp.int32)
    incl = _cumsum_lanes(cond_i, n)
    rows = jnp.where(cond, incl - cond_i + 1, n + 1)

    i = pl.program_id(0)
    row_ids = i * row_block + jax.lax.broadcasted_iota(
        jnp.int32, (row_block, n), 0
    )
    out = jnp.where(row_ids == rows, err, 0.0)
    out = jnp.where(row_ids == 0, center, out)
    out_ref[...] = out


@jax.jit
def kernel(x):
    C, H, W = x.shape
    n = C * H * W
    n_rows = n + 1
    grid = pl.cdiv(n_rows, ROW_BLOCK)
    xf = x.reshape(1, n)
    z = pl.pallas_call(
        functools.partial(_zono_kernel, n=n, row_block=ROW_BLOCK),
        grid=(grid,),
        in_specs=[pl.BlockSpec((1, n), lambda i: (0, 0))],
        out_specs=pl.BlockSpec((ROW_BLOCK, n), lambda i: (i, 0)),
        out_shape=jax.ShapeDtypeStruct((n_rows, n), x.dtype),
    )(xf)
    return z.reshape(n_rows, C, H, W)
